# Initial kernel scaffold; baseline (speedup 1.0000x reference)
import jax
import jax.numpy as jnp
from jax import lax
from jax.experimental import pallas as pl
from jax.experimental.pallas import tpu as pltpu
from jax.experimental.pallas import tpu_sc as plsc

NC = 2
NS = 16
L = 16
NW = NC * NS
CH = 10000


def _sc_edge_filter(si_hbm, src_hbm, dst_hbm, attr_hbm, x_hbm,
                    partials_hbm, xsi_hbm,
                    si_v, src_v, dst_v, attr_v, rows_v, w_scr, acc, sem):
    E = src_hbm.shape[0]
    epw = E // NW
    n_chunks = epw // CH
    nv = CH // L

    wid = lax.axis_index("s") * NC + lax.axis_index("c")
    base = wid * epw

    acc[...] = jnp.zeros((L,), jnp.float32)
    pltpu.sync_copy(si_hbm, si_v)
    si_vec = si_v[...]

    def chunk_body(ci, _):
        off = base + ci * CH
        pltpu.sync_copy(src_hbm.at[pl.ds(off, CH)], src_v)
        pltpu.sync_copy(dst_hbm.at[pl.ds(off, CH)], dst_v)
        pltpu.sync_copy(attr_hbm.at[pl.ds(off, CH)], attr_v)

        def vec_body(v, _):
            dvec = dst_v[pl.ds(v * L, L)]
            mask = dvec == si_vec

            @pl.when(jnp.any(mask))
            def _():
                wv = jnp.where(mask, attr_v[pl.ds(v * L, L)], 0.0)
                svec = src_v[pl.ds(v * L, L)]
                pltpu.async_copy(x_hbm.at[svec], rows_v, sem).wait()
                w_scr[...] = wv
                for lane in range(L):
                    acc[...] = acc[...] + w_scr[lane] * rows_v[lane, :]

            return 0

        lax.fori_loop(0, nv, vec_body, 0)
        return 0

    lax.fori_loop(0, n_chunks, chunk_body, 0)

    pltpu.sync_copy(acc, partials_hbm.at[wid])

    @pl.when(wid == 0)
    def _():
        si0 = si_v[0]
        pltpu.sync_copy(x_hbm.at[pl.ds(si0, 1)], rows_v.at[pl.ds(0, 1)])
        pltpu.sync_copy(rows_v.at[pl.ds(0, 1)], xsi_hbm)


def _sc_call(si_vec, src, dst, attr, x):
    mesh = plsc.VectorSubcoreMesh(
        core_axis_name="c", subcore_axis_name="s", num_cores=NC, num_subcores=NS)
    return pl.kernel(
        _sc_edge_filter,
        out_type=(
            jax.ShapeDtypeStruct((NW, L), jnp.float32),
            jax.ShapeDtypeStruct((1, L), jnp.float32),
        ),
        mesh=mesh,
        scratch_types=(
            pltpu.VMEM((L,), jnp.int32),
            pltpu.VMEM((CH,), jnp.int32),
            pltpu.VMEM((CH,), jnp.int32),
            pltpu.VMEM((CH,), jnp.float32),
            pltpu.VMEM((L, L), jnp.float32),
            pltpu.VMEM((L,), jnp.float32),
            pltpu.VMEM((L,), jnp.float32),
            pltpu.SemaphoreType.DMA,
        ),
    )(si_vec, src, dst, attr, x)


def _tc_dense(partials, xsi, h0, c0, Wn, Ws, bg, WihT, WhhT, bsum,
              W1, b1, W2, b2, xo_out, h_out, c_out):
    agg = jnp.sum(partials[...], axis=0, keepdims=True)
    xr = xsi[...]
    xg = agg @ Wn[...] + xr @ Ws[...] + bg[...]
    xg = jnp.maximum(xg, 0.0)
    gates = xg @ WihT[...] + h0[...] @ WhhT[...] + bsum[...]
    i = jax.nn.sigmoid(gates[:, 0:64])
    f = jax.nn.sigmoid(gates[:, 64:128])
    g = jnp.tanh(gates[:, 128:192])
    o = jax.nn.sigmoid(gates[:, 192:256])
    c1 = f * c0[...] + i * g
    h1 = o * jnp.tanh(c1)
    xcat = jnp.concatenate([xg, h1], axis=1)
    xo = xcat @ W1[...] + b1[...]
    xo_out[...] = xo @ W2[...] + b2[...]
    h_out[...] = h1
    c_out[...] = c1


def kernel(x, edge_index, edge_attr, h, c, state_index,
           W_neigh, W_self, b_gnn, W_ih, W_hh, b_ih, b_hh, W1, b1, W2, b2):
    E = edge_index.shape[1]
    H = W_neigh.shape[1]

    src = edge_index[0]
    dst = edge_index[1]
    attr = edge_attr.reshape(E)
    si_vec = jnp.full((L,), jnp.asarray(state_index, jnp.int32), jnp.int32)

    partials, xsi = _sc_call(si_vec, src, dst, attr, x)

    h0 = h.reshape(1, H)
    c0 = c.reshape(1, H)
    bsum = (b_ih + b_hh).reshape(1, 4 * H)

    xo, h1, c1 = pl.pallas_call(
        _tc_dense,
        out_shape=(
            jax.ShapeDtypeStruct((1, 4), jnp.float32),
            jax.ShapeDtypeStruct((1, H), jnp.float32),
            jax.ShapeDtypeStruct((1, H), jnp.float32),
        ),
    )(partials, xsi, h0, c0, W_neigh, W_self, b_gnn.reshape(1, H),
      W_ih.T, W_hh.T, bsum, W1, b1.reshape(1, -1), W2, b2.reshape(1, -1))

    return (xo, h1.reshape(1, 1, H), c1.reshape(1, 1, H))


# speedup vs baseline: 81.7440x; 81.7440x over previous
import jax
import jax.numpy as jnp
from jax import lax
from jax.experimental import pallas as pl
from jax.experimental.pallas import tpu as pltpu
from jax.experimental.pallas import tpu_sc as plsc

NC = 2
NS = 16
L = 16
NW = NC * NS
CH = 10000


def _sc_edge_filter(si_hbm, src_hbm, dst_hbm, attr_hbm, x_hbm,
                    partials_hbm, xsi_hbm,
                    si_v, src_v, dst_v, attr_v, rows_v, acc, sem):
    E = src_hbm.shape[0]
    epw = E // NW
    n_chunks = epw // CH
    nv = CH // L

    wid = lax.axis_index("s") * NC + lax.axis_index("c")
    base = wid * epw

    acc[...] = jnp.zeros((L,), jnp.float32)
    pltpu.sync_copy(si_hbm, si_v)
    si_vec = si_v[...]

    def chunk_body(ci, _):
        off = base + ci * CH
        pltpu.sync_copy(src_hbm.at[pl.ds(off, CH)], src_v)
        pltpu.sync_copy(dst_hbm.at[pl.ds(off, CH)], dst_v)
        pltpu.sync_copy(attr_hbm.at[pl.ds(off, CH)], attr_v)

        def vec_body(v, _):
            dvec = dst_v[pl.ds(v * L, L)]
            mask = dvec == si_vec
            nmatch = jnp.sum(jnp.where(mask, 1, 0))

            @pl.when(nmatch > 0)
            def _():
                wv = jnp.where(mask, attr_v[pl.ds(v * L, L)], 0.0)
                svec = src_v[pl.ds(v * L, L)]
                pltpu.async_copy(x_hbm.at[svec], rows_v, sem).wait()
                for lane in range(L):
                    acc[...] = acc[...] + wv[lane] * rows_v[lane, :]

            return 0

        lax.fori_loop(0, nv, vec_body, 0)
        return 0

    lax.fori_loop(0, n_chunks, chunk_body, 0)

    pltpu.sync_copy(acc, partials_hbm.at[wid])

    @pl.when(wid == 0)
    def _():
        si0 = si_vec[0]
        pltpu.sync_copy(x_hbm.at[pl.ds(si0, 1)], rows_v.at[pl.ds(0, 1)])
        pltpu.sync_copy(rows_v.at[pl.ds(0, 1)], xsi_hbm)


def _sc_call(si_vec, src, dst, attr, x):
    mesh = plsc.VectorSubcoreMesh(
        core_axis_name="c", subcore_axis_name="s", num_cores=NC, num_subcores=NS)
    return pl.kernel(
        _sc_edge_filter,
        out_type=(
            jax.ShapeDtypeStruct((NW, L), jnp.float32),
            jax.ShapeDtypeStruct((1, L), jnp.float32),
        ),
        mesh=mesh,
        compiler_params=pltpu.CompilerParams(
            needs_layout_passes=False, use_tc_tiling_on_sc=False),
        scratch_types=(
            pltpu.VMEM((L,), jnp.int32),
            pltpu.VMEM((CH,), jnp.int32),
            pltpu.VMEM((CH,), jnp.int32),
            pltpu.VMEM((CH,), jnp.float32),
            pltpu.VMEM((L, L), jnp.float32),
            pltpu.VMEM((L,), jnp.float32),
            pltpu.SemaphoreType.DMA,
        ),
    )(si_vec, src, dst, attr, x)


def _tc_dense(partials, xsi, h0, c0, Wn, Ws, bg, WihT, WhhT, bsum,
              W1, b1, W2, b2, xo_out, h_out, c_out):
    dot = lambda a, b: jax.lax.dot(a, b, precision=jax.lax.Precision.DEFAULT,
                                   preferred_element_type=jnp.float32)
    agg = jnp.sum(partials[...], axis=0, keepdims=True)
    xr = xsi[...]
    xg = dot(agg, Wn[...]) + dot(xr, Ws[...]) + bg[...]
    xg = jnp.maximum(xg, 0.0)
    gates = dot(xg, WihT[...]) + dot(h0[...], WhhT[...]) + bsum[...]
    i = jax.nn.sigmoid(gates[:, 0:64])
    f = jax.nn.sigmoid(gates[:, 64:128])
    g = jnp.tanh(gates[:, 128:192])
    o = jax.nn.sigmoid(gates[:, 192:256])
    c1 = f * c0[...] + i * g
    h1 = o * jnp.tanh(c1)
    xcat = jnp.concatenate([xg, h1], axis=1)
    xo = dot(xcat, W1[...]) + b1[...]
    xo_out[...] = dot(xo, W2[...]) + b2[...]
    h_out[...] = h1
    c_out[...] = c1


def kernel(x, edge_index, edge_attr, h, c, state_index,
           W_neigh, W_self, b_gnn, W_ih, W_hh, b_ih, b_hh, W1, b1, W2, b2):
    E = edge_index.shape[1]
    H = W_neigh.shape[1]

    src = edge_index[0]
    dst = edge_index[1]
    attr = edge_attr.reshape(E)
    si_vec = jnp.full((L,), jnp.asarray(state_index, jnp.int32), jnp.int32)

    partials, xsi = _sc_call(si_vec, src, dst, attr, x)

    h0 = h.reshape(1, H)
    c0 = c.reshape(1, H)
    bsum = (b_ih + b_hh).reshape(1, 4 * H)

    xo, h1, c1 = pl.pallas_call(
        _tc_dense,
        out_shape=(
            jax.ShapeDtypeStruct((1, 4), jnp.float32),
            jax.ShapeDtypeStruct((1, H), jnp.float32),
            jax.ShapeDtypeStruct((1, H), jnp.float32),
        ),
    )(partials, xsi, h0, c0, W_neigh, W_self, b_gnn.reshape(1, H),
      W_ih.T, W_hh.T, bsum, W1, b1.reshape(1, -1), W2, b2.reshape(1, -1))

    return (xo, h1.reshape(1, 1, H), c1.reshape(1, 1, H))


# speedup vs baseline: 195.4846x; 2.3914x over previous
import jax
import jax.numpy as jnp
from jax import lax
from jax.experimental import pallas as pl
from jax.experimental.pallas import tpu as pltpu
from jax.experimental.pallas import tpu_sc as plsc

NC = 2
NS = 16
L = 16
NW = NC * NS
CH = 20000


G = 10


def _sc_edge_filter(si_hbm, src_hbm, dst_hbm, attr_hbm, x_hbm,
                    partials_hbm, xsi_hbm,
                    si_v, dst_v0, dst_v1, src_g, attr_g, rows_v, acc,
                    sem0, sem1, semg):
    E = src_hbm.shape[0]
    epw = E // NW
    n_chunks = epw // CH
    ng = CH // (G * L)

    wid = lax.axis_index("s") * NC + lax.axis_index("c")
    base = wid * epw

    acc[...] = jnp.zeros((L,), jnp.float32)
    pltpu.sync_copy(si_hbm, si_v)
    si_vec = si_v[...]

    bufs = (dst_v0, dst_v1)
    sems = (sem0, sem1)
    pltpu.async_copy(dst_hbm.at[pl.ds(base, CH)], bufs[0], sems[0])

    for ci in range(n_chunks):
        buf = bufs[ci % 2]
        pltpu.make_async_copy(dst_hbm.at[pl.ds(base + ci * CH, CH)],
                              buf, sems[ci % 2]).wait()
        if ci + 1 < n_chunks:
            pltpu.async_copy(dst_hbm.at[pl.ds(base + (ci + 1) * CH, CH)],
                             bufs[(ci + 1) % 2], sems[(ci + 1) % 2])

        def group_body(g, _, buf=buf, ci=ci):
            gbase = g * (G * L)
            hits = jnp.zeros((L,), jnp.int32)
            for v in range(G):
                dvec = buf[pl.ds(gbase + v * L, L)]
                hits = hits + jnp.where(dvec == si_vec, 1, 0)

            @pl.when(jnp.sum(hits) > 0)
            def _():
                for v in range(G):
                    dvec = buf[pl.ds(gbase + v * L, L)]
                    mask = dvec == si_vec
                    nm = jnp.sum(jnp.where(mask, 1, 0))

                    @pl.when(nm > 0)
                    def _():
                        eoff = base + ci * CH + gbase + v * L
                        cp_s = pltpu.async_copy(
                            src_hbm.at[pl.ds(eoff, L)], src_g, semg)
                        cp_a = pltpu.async_copy(
                            attr_hbm.at[pl.ds(eoff, L)], attr_g, semg)
                        cp_s.wait()
                        cp_a.wait()
                        wv = jnp.where(mask, attr_g[...], 0.0)
                        pltpu.async_copy(
                            x_hbm.at[src_g[...]], rows_v, semg).wait()
                        for lane in range(L):
                            acc[...] = acc[...] + wv[lane] * rows_v[lane, :]

            return 0

        lax.fori_loop(0, ng, group_body, 0)

    pltpu.sync_copy(acc, partials_hbm.at[wid])

    @pl.when(wid == 0)
    def _():
        si0 = si_vec[0]
        pltpu.sync_copy(x_hbm.at[pl.ds(si0, 1)], rows_v.at[pl.ds(0, 1)])
        pltpu.sync_copy(rows_v.at[pl.ds(0, 1)], xsi_hbm)


def _sc_call(si_vec, src, dst, attr, x):
    mesh = plsc.VectorSubcoreMesh(
        core_axis_name="c", subcore_axis_name="s", num_cores=NC, num_subcores=NS)
    return pl.kernel(
        _sc_edge_filter,
        out_type=(
            jax.ShapeDtypeStruct((NW, L), jnp.float32),
            jax.ShapeDtypeStruct((1, L), jnp.float32),
        ),
        mesh=mesh,
        compiler_params=pltpu.CompilerParams(
            needs_layout_passes=False, use_tc_tiling_on_sc=False),
        scratch_types=(
            pltpu.VMEM((L,), jnp.int32),
            pltpu.VMEM((CH,), jnp.int32),
            pltpu.VMEM((CH,), jnp.int32),
            pltpu.VMEM((L,), jnp.int32),
            pltpu.VMEM((L,), jnp.float32),
            pltpu.VMEM((L, L), jnp.float32),
            pltpu.VMEM((L,), jnp.float32),
            pltpu.SemaphoreType.DMA,
            pltpu.SemaphoreType.DMA,
            pltpu.SemaphoreType.DMA,
        ),
    )(si_vec, src, dst, attr, x)


def _tc_dense(partials, xsi, h0, c0, Wn, Ws, bg, WihT, WhhT, bsum,
              W1, b1, W2, b2, xo_out, h_out, c_out):
    dot = lambda a, b: jax.lax.dot(a, b, precision=jax.lax.Precision.DEFAULT,
                                   preferred_element_type=jnp.float32)
    agg = jnp.sum(partials[...], axis=0, keepdims=True)
    xr = xsi[...]
    xg = dot(agg, Wn[...]) + dot(xr, Ws[...]) + bg[...]
    xg = jnp.maximum(xg, 0.0)
    gates = dot(xg, WihT[...]) + dot(h0[...], WhhT[...]) + bsum[...]
    i = jax.nn.sigmoid(gates[:, 0:64])
    f = jax.nn.sigmoid(gates[:, 64:128])
    g = jnp.tanh(gates[:, 128:192])
    o = jax.nn.sigmoid(gates[:, 192:256])
    c1 = f * c0[...] + i * g
    h1 = o * jnp.tanh(c1)
    xcat = jnp.concatenate([xg, h1], axis=1)
    xo = dot(xcat, W1[...]) + b1[...]
    xo_out[...] = dot(xo, W2[...]) + b2[...]
    h_out[...] = h1
    c_out[...] = c1


def kernel(x, edge_index, edge_attr, h, c, state_index,
           W_neigh, W_self, b_gnn, W_ih, W_hh, b_ih, b_hh, W1, b1, W2, b2):
    E = edge_index.shape[1]
    H = W_neigh.shape[1]

    src = edge_index[0]
    dst = edge_index[1]
    attr = edge_attr.reshape(E)
    si_vec = jnp.full((L,), jnp.asarray(state_index, jnp.int32), jnp.int32)

    partials, xsi = _sc_call(si_vec, src, dst, attr, x)

    h0 = h.reshape(1, H)
    c0 = c.reshape(1, H)
    bsum = (b_ih + b_hh).reshape(1, 4 * H)

    xo, h1, c1 = pl.pallas_call(
        _tc_dense,
        out_shape=(
            jax.ShapeDtypeStruct((1, 4), jnp.float32),
            jax.ShapeDtypeStruct((1, H), jnp.float32),
            jax.ShapeDtypeStruct((1, H), jnp.float32),
        ),
    )(partials, xsi, h0, c0, W_neigh, W_self, b_gnn.reshape(1, H),
      W_ih.T, W_hh.T, bsum, W1, b1.reshape(1, -1), W2, b2.reshape(1, -1))

    return (xo, h1.reshape(1, 1, H), c1.reshape(1, 1, H))


# speedup vs baseline: 209.9605x; 1.0741x over previous
import jax
import jax.numpy as jnp
from jax import lax
from jax.experimental import pallas as pl
from jax.experimental.pallas import tpu as pltpu
from jax.experimental.pallas import tpu_sc as plsc

NC = 2
NS = 16
L = 16
NW = NC * NS
CH = 20000


G = 10


def _sc_edge_filter(si_hbm, ei_hbm, attr_hbm, x_hbm,
                    partials_hbm, xsi_hbm,
                    si_v, dst_v0, dst_v1, src_g, attr_g, rows_v, acc,
                    sem0, sem1, semg):
    E = ei_hbm.shape[1]
    epw = E // NW
    n_chunks = epw // CH
    ng = CH // (G * L)

    wid = lax.axis_index("s") * NC + lax.axis_index("c")
    base = wid * epw

    acc[...] = jnp.zeros((L,), jnp.float32)
    pltpu.sync_copy(si_hbm, si_v)
    si_vec = si_v[...]

    bufs = (dst_v0, dst_v1)
    sems = (sem0, sem1)
    pltpu.async_copy(ei_hbm.at[1, pl.ds(base, CH)], bufs[0], sems[0])

    for ci in range(n_chunks):
        buf = bufs[ci % 2]
        pltpu.make_async_copy(ei_hbm.at[1, pl.ds(base + ci * CH, CH)],
                              buf, sems[ci % 2]).wait()
        if ci + 1 < n_chunks:
            pltpu.async_copy(ei_hbm.at[1, pl.ds(base + (ci + 1) * CH, CH)],
                             bufs[(ci + 1) % 2], sems[(ci + 1) % 2])

        def group_body(g, _, buf=buf, ci=ci):
            gbase = g * (G * L)
            hits = jnp.zeros((L,), jnp.int32)
            for v in range(G):
                dvec = buf[pl.ds(gbase + v * L, L)]
                hits = hits + jnp.where(dvec == si_vec, 1, 0)

            @pl.when(jnp.sum(hits) > 0)
            def _():
                for v in range(G):
                    dvec = buf[pl.ds(gbase + v * L, L)]
                    mask = dvec == si_vec
                    nm = jnp.sum(jnp.where(mask, 1, 0))

                    @pl.when(nm > 0)
                    def _():
                        eoff = base + ci * CH + gbase + v * L
                        cp_s = pltpu.async_copy(
                            ei_hbm.at[0, pl.ds(eoff, L)], src_g, semg)
                        cp_a = pltpu.async_copy(
                            attr_hbm.at[pl.ds(eoff, L)], attr_g, semg)
                        cp_s.wait()
                        cp_a.wait()
                        wv = jnp.where(mask, attr_g[...], 0.0)
                        pltpu.async_copy(
                            x_hbm.at[src_g[...]], rows_v, semg).wait()
                        for lane in range(L):
                            acc[...] = acc[...] + wv[lane] * rows_v[lane, :]

            return 0

        lax.fori_loop(0, ng, group_body, 0)

    pltpu.sync_copy(acc, partials_hbm.at[wid])

    @pl.when(wid == 0)
    def _():
        si0 = si_vec[0]
        pltpu.sync_copy(x_hbm.at[pl.ds(si0, 1)], rows_v.at[pl.ds(0, 1)])
        pltpu.sync_copy(rows_v.at[pl.ds(0, 1)], xsi_hbm)


def _sc_call(si_vec, ei, attr, x):
    mesh = plsc.VectorSubcoreMesh(
        core_axis_name="c", subcore_axis_name="s", num_cores=NC, num_subcores=NS)
    return pl.kernel(
        _sc_edge_filter,
        out_type=(
            jax.ShapeDtypeStruct((NW, L), jnp.float32),
            jax.ShapeDtypeStruct((1, L), jnp.float32),
        ),
        mesh=mesh,
        compiler_params=pltpu.CompilerParams(
            needs_layout_passes=False, use_tc_tiling_on_sc=False),
        scratch_types=(
            pltpu.VMEM((L,), jnp.int32),
            pltpu.VMEM((CH,), jnp.int32),
            pltpu.VMEM((CH,), jnp.int32),
            pltpu.VMEM((L,), jnp.int32),
            pltpu.VMEM((L,), jnp.float32),
            pltpu.VMEM((L, L), jnp.float32),
            pltpu.VMEM((L,), jnp.float32),
            pltpu.SemaphoreType.DMA,
            pltpu.SemaphoreType.DMA,
            pltpu.SemaphoreType.DMA,
        ),
    )(si_vec, ei, attr, x)


def _tc_dense(partials, xsi, h0, c0, Wn, Ws, bg, WihT, WhhT, bsum,
              W1, b1, W2, b2, xo_out, h_out, c_out):
    dot = lambda a, b: jax.lax.dot(a, b, precision=jax.lax.Precision.DEFAULT,
                                   preferred_element_type=jnp.float32)
    agg = jnp.sum(partials[...], axis=0, keepdims=True)
    xr = xsi[...]
    xg = dot(agg, Wn[...]) + dot(xr, Ws[...]) + bg[...]
    xg = jnp.maximum(xg, 0.0)
    gates = dot(xg, WihT[...]) + dot(h0[...], WhhT[...]) + bsum[...]
    i = jax.nn.sigmoid(gates[:, 0:64])
    f = jax.nn.sigmoid(gates[:, 64:128])
    g = jnp.tanh(gates[:, 128:192])
    o = jax.nn.sigmoid(gates[:, 192:256])
    c1 = f * c0[...] + i * g
    h1 = o * jnp.tanh(c1)
    xcat = jnp.concatenate([xg, h1], axis=1)
    xo = dot(xcat, W1[...]) + b1[...]
    xo_out[...] = dot(xo, W2[...]) + b2[...]
    h_out[...] = h1
    c_out[...] = c1


def kernel(x, edge_index, edge_attr, h, c, state_index,
           W_neigh, W_self, b_gnn, W_ih, W_hh, b_ih, b_hh, W1, b1, W2, b2):
    E = edge_index.shape[1]
    H = W_neigh.shape[1]

    attr = edge_attr.reshape(E)
    si_vec = jnp.full((L,), jnp.asarray(state_index, jnp.int32), jnp.int32)

    partials, xsi = _sc_call(si_vec, edge_index, attr, x)

    h0 = h.reshape(1, H)
    c0 = c.reshape(1, H)
    bsum = (b_ih + b_hh).reshape(1, 4 * H)

    xo, h1, c1 = pl.pallas_call(
        _tc_dense,
        out_shape=(
            jax.ShapeDtypeStruct((1, 4), jnp.float32),
            jax.ShapeDtypeStruct((1, H), jnp.float32),
            jax.ShapeDtypeStruct((1, H), jnp.float32),
        ),
    )(partials, xsi, h0, c0, W_neigh, W_self, b_gnn.reshape(1, H),
      W_ih.T, W_hh.T, bsum, W1, b1.reshape(1, -1), W2, b2.reshape(1, -1))

    return (xo, h1.reshape(1, 1, H), c1.reshape(1, 1, H))
